# baseline (speedup 1.0000x reference)
import functools

import jax
import jax.numpy as jnp
from jax import lax
from jax.experimental import pallas as pl
from jax.experimental.pallas import tpu as pltpu
from jax.experimental.pallas import tpu_sc as plsc

B = 16384
F = 512
D = 128
T = 819200
NC = 2
NS = 16
NW = NC * NS
RC = 32
NCH = B // RC
CPW = NCH // NW
TB = 2048
NOFF = 528

_mesh = plsc.VectorSubcoreMesh(
    core_axis_name="c", subcore_axis_name="s", num_cores=NC, num_subcores=NS
)
_sc_params = pltpu.CompilerParams(needs_layout_passes=False)


@functools.partial(
    pl.kernel,
    out_type=jax.ShapeDtypeStruct((B, D), jnp.float32),
    mesh=_mesh,
    compiler_params=_sc_params,
    scratch_types=[
        pltpu.VMEM((2, 128), jnp.int32),
        pltpu.VMEM((2, 128, D), jnp.float32),
        pltpu.SemaphoreType.DMA((2,)),
    ],
)
def _gather_h(user_hbm, h_tab_hbm, out_hbm, idx_v, rows_v, sem_g):
    w = lax.axis_index("s") * NC + lax.axis_index("c")
    nq = (B // NW) // 128

    def start(q, slot):
        base = w * (B // NW) + q * 128
        pltpu.sync_copy(user_hbm.at[pl.ds(base, 128)], idx_v.at[slot])
        pltpu.async_copy(h_tab_hbm.at[idx_v.at[slot]], rows_v.at[slot],
                         sem_g.at[slot])

    start(0, 0)
    for q in range(nq):
        slot = q % 2
        if q + 1 < nq:
            start(q + 1, 1 - slot)
        base = w * (B // NW) + q * 128
        pltpu.make_async_copy(h_tab_hbm.at[idx_v.at[slot]], rows_v.at[slot],
                              sem_g.at[slot]).wait()
        pltpu.sync_copy(rows_v.at[slot], out_hbm.at[pl.ds(base, 128), :])


def _mm_body(h_ref, g_ref, o_ref):
    o_ref[...] = lax.dot_general(
        h_ref[...], g_ref[...],
        (((1,), (1,)), ((), ())),
        preferred_element_type=jnp.float32,
    )


@functools.partial(
    pl.kernel,
    out_type=jax.ShapeDtypeStruct((NW, 16), jnp.float32),
    mesh=_mesh,
    compiler_params=_sc_params,
    scratch_types=[
        pltpu.VMEM((32,), jnp.int32),
        pltpu.VMEM((2, RC), jnp.int32),
        pltpu.VMEM((2, RC, F), jnp.float32),
        pltpu.VMEM((2, RC, F), jnp.float32),
        pltpu.VMEM((TB,), jnp.int32),
        pltpu.VMEM((TB,), jnp.int32),
        pltpu.VMEM((16,), jnp.float32),
        pltpu.SemaphoreType.DMA((2,)),
        pltpu.SemaphoreType.DMA((2,)),
    ],
)
def _main_sc(k_tab_hbm, z_hbm, user_hbm, seg_hbm, feat_hbm, offs_hbm, out_hbm,
             offs_v, uidx_v, krows2_v, zrows2_v, sbuf_v, fbuf_v, acc_v,
             sem_kz, sem_i):
    w = lax.axis_index("s") * NC + lax.axis_index("c")
    pltpu.sync_copy(offs_hbm.at[pl.ds(w * CPW, 32)], offs_v)
    o_lo = offs_v[pl.ds(0, 16)]
    o_hi = offs_v[pl.ds(16, 16)]
    lanes = lax.iota(jnp.int32, 16)

    def off(j):
        return o_lo[j] if j < 16 else o_hi[j - 16]

    def start_chunk(j, slot):
        r0 = (w * CPW + j) * RC
        pltpu.sync_copy(user_hbm.at[pl.ds(r0, RC)], uidx_v.at[slot])
        pltpu.async_copy(k_tab_hbm.at[uidx_v.at[slot]], krows2_v.at[slot],
                         sem_kz.at[slot])
        pltpu.async_copy(z_hbm.at[pl.ds(r0, RC), :], zrows2_v.at[slot],
                         sem_kz.at[slot])

    start_chunk(0, 0)
    acc = jnp.zeros((16,), jnp.float32)
    for j in range(CPW):
        slot = j % 2
        if j + 1 < CPW:
            start_chunk(j + 1, 1 - slot)
        krows_v = krows2_v.at[slot]
        zrows_v = zrows2_v.at[slot]
        t0 = off(j)
        t1 = off(j + 1)
        r0 = (w * CPW + j) * RC
        pltpu.make_async_copy(k_tab_hbm.at[uidx_v.at[slot]],
                              krows2_v.at[slot], sem_kz.at[slot]).wait()
        pltpu.make_async_copy(z_hbm.at[pl.ds(r0, RC), :],
                              zrows2_v.at[slot], sem_kz.at[slot]).wait()
        lo = r0 * F
        hi = lo + RC * F
        t0a = t0 & (-8)
        nb = (t1 - t0a + TB - 1) // TB

        def batch_body(bi, acc):
            wm = t0a + bi * TB
            bs = pl.multiple_of(jnp.minimum(wm, T - TB), 8)
            ci = pltpu.async_copy(seg_hbm.at[pl.ds(bs, TB)], sbuf_v,
                                  sem_i.at[0])
            cf = pltpu.async_copy(feat_hbm.at[pl.ds(bs, TB)], fbuf_v,
                                  sem_i.at[1])
            ci.wait()
            cf.wait()

            def grp(i, acc):
                sv = sbuf_v[pl.ds(i * 16, 16)]
                fv = fbuf_v[pl.ds(i * 16, 16)]
                cv = (sv << 9) | fv
                pos = lanes + (bs + i * 16)
                m = (cv >= lo) & (cv < hi)
                m = m & (pos >= wm) & (pos < wm + TB)
                loc = cv - lo
                r = (loc >> 9) & (RC - 1)
                col = loc & (F - 1)
                kv = plsc.load_gather(krows_v, [r, col], mask=m)
                zv = plsc.load_gather(zrows_v, [r, col], mask=m)
                return acc + jnp.where(m, kv * zv, 0.0)

            return lax.fori_loop(0, TB // 16, grp, acc)

        acc = lax.fori_loop(0, nb, batch_body, acc)

    acc_v[...] = acc
    pltpu.sync_copy(acc_v, out_hbm.at[w])


def kernel(H, G, K, user, item, feat_idx, segment_ids):
    del item
    queries = jnp.minimum(
        jnp.arange(0, NOFF * RC, RC, dtype=jnp.int32), B
    )
    offs = jnp.searchsorted(segment_ids, queries, side="left").astype(jnp.int32)

    h_u = _gather_h(user, H)

    BM = 1024
    z = pl.pallas_call(
        _mm_body,
        grid=(B // BM,),
        in_specs=[
            pl.BlockSpec((BM, D), lambda i: (i, 0)),
            pl.BlockSpec((F, D), lambda i: (0, 0)),
        ],
        out_specs=pl.BlockSpec((BM, F), lambda i: (i, 0)),
        out_shape=jax.ShapeDtypeStruct((B, F), jnp.float32),
    )(h_u, G)

    partials = _main_sc(K, z, user, segment_ids, feat_idx, offs)
    return jnp.sum(partials)


# speedup vs baseline: 1.1257x; 1.1257x over previous
import functools

import jax
import jax.numpy as jnp
from jax import lax
from jax.experimental import pallas as pl
from jax.experimental.pallas import tpu as pltpu
from jax.experimental.pallas import tpu_sc as plsc

B = 16384
F = 512
D = 128
T = 819200
NC = 2
NS = 16
NW = NC * NS
RC = 64
RSH = 6
NCH = B // RC
CPW = NCH // NW
TB = 2048
TPW = T // NW
SSB = 1600
NSB = TPW // SSB
TABW = 272

_mesh = plsc.VectorSubcoreMesh(
    core_axis_name="c", subcore_axis_name="s", num_cores=NC, num_subcores=NS
)
_sc_params = pltpu.CompilerParams(needs_layout_passes=False)


@functools.partial(
    pl.kernel,
    out_type=(
        jax.ShapeDtypeStruct((B, D), jnp.float32),
        jax.ShapeDtypeStruct((NW * 2 * TABW,), jnp.int32),
    ),
    mesh=_mesh,
    compiler_params=_sc_params,
    scratch_types=[
        pltpu.VMEM((2, 128), jnp.int32),
        pltpu.VMEM((2, 128, D), jnp.float32),
        pltpu.VMEM((SSB,), jnp.int32),
        pltpu.VMEM((16,), jnp.int32),
        pltpu.VMEM((TABW,), jnp.int32),
        pltpu.VMEM((TABW,), jnp.int32),
        pltpu.SemaphoreType.DMA((2,)),
    ],
)
def _gather_h(user_hbm, h_tab_hbm, seg_hbm, out_hbm, tabs_hbm,
              idx_v, rows_v, scan_v, pv_v, beg_v, end_v, sem_g):
    w = lax.axis_index("s") * NC + lax.axis_index("c")
    lanes = lax.iota(jnp.int32, 16)
    shl = jnp.maximum(lanes - 1, 0)

    nq = (B // NW) // 128

    def start(q, slot):
        base = w * (B // NW) + q * 128
        pltpu.sync_copy(user_hbm.at[pl.ds(base, 128)], idx_v.at[slot])
        pltpu.async_copy(h_tab_hbm.at[idx_v.at[slot]], rows_v.at[slot],
                         sem_g.at[slot])

    start(0, 0)
    for q in range(nq):
        slot = q % 2
        if q + 1 < nq:
            start(q + 1, 1 - slot)
        base = w * (B // NW) + q * 128
        pltpu.make_async_copy(h_tab_hbm.at[idx_v.at[slot]], rows_v.at[slot],
                              sem_g.at[slot]).wait()
        pltpu.sync_copy(rows_v.at[slot], out_hbm.at[pl.ds(base, 128), :])

    zv = jnp.zeros((16,), jnp.int32)
    for i in range(TABW // 16):
        beg_v[pl.ds(i * 16, 16)] = zv
        end_v[pl.ds(i * 16, 16)] = zv

    p0 = w * TPW
    poff = pl.multiple_of(jnp.maximum(p0 - 16, 0), 16)
    pltpu.sync_copy(seg_hbm.at[pl.ds(poff, 16)], pv_v)
    pvv = pv_v[...]
    prev0 = jnp.where(w == 0, pvv[0], pvv[15]) >> RSH

    prev = prev0
    for bi in range(NSB):
        bs = pl.multiple_of(p0 + bi * SSB, 8)
        pltpu.sync_copy(seg_hbm.at[pl.ds(bs, SSB)], scan_v)

        def sgrp(g, prev):
            sv = scan_v[pl.ds(g * 16, 16)]
            chv = sv >> RSH
            shifted = chv.at[shl].get(mode="promise_in_bounds")
            prevv = jnp.broadcast_to(prev, (16,))
            left = jnp.where(lanes == 0, prevv, shifted)
            mcross = chv != left
            pos = (bs + g * 16) + lanes
            plsc.store_scatter(beg_v, [chv], pos, mask=mcross)
            plsc.store_scatter(end_v, [left], pos, mask=mcross)
            return chv[15]

        prev = lax.fori_loop(0, SSB // 16, sgrp, prev)

    @pl.when(w == NW - 1)
    def _():
        pv = jnp.broadcast_to(prev, (16,))
        tv = jnp.broadcast_to(jnp.int32(T), (16,))
        plsc.store_scatter(end_v, [pv], tv, mask=lanes == 0)

    pltpu.sync_copy(beg_v, tabs_hbm.at[pl.ds(w * 2 * TABW, TABW)])
    pltpu.sync_copy(end_v, tabs_hbm.at[pl.ds((w * 2 + 1) * TABW, TABW)])


def _mm_body(h_ref, g_ref, o_ref):
    o_ref[...] = lax.dot_general(
        h_ref[...], g_ref[...],
        (((1,), (1,)), ((), ())),
        preferred_element_type=jnp.float32,
    )


@functools.partial(
    pl.kernel,
    out_type=jax.ShapeDtypeStruct((NW, 16), jnp.float32),
    mesh=_mesh,
    compiler_params=_sc_params,
    scratch_types=[
        pltpu.VMEM((16,), jnp.int32),
        pltpu.VMEM((16,), jnp.int32),
        pltpu.VMEM((RC,), jnp.int32),
        pltpu.VMEM((RC, F), jnp.float32),
        pltpu.VMEM((RC, F), jnp.float32),
        pltpu.VMEM((TB,), jnp.int32),
        pltpu.VMEM((TB,), jnp.int32),
        pltpu.VMEM((16,), jnp.float32),
        pltpu.SemaphoreType.DMA,
        pltpu.SemaphoreType.DMA,
        pltpu.SemaphoreType.DMA((2,)),
    ],
)
def _main_sc(k_tab_hbm, z_hbm, user_hbm, seg_hbm, feat_hbm, begm_hbm,
             endm_hbm, out_hbm,
             bwin_v, ewin_v, uidx_v, krows_v, zrows_v, sbuf_v, fbuf_v,
             acc_v, sem_k, sem_z, sem_i):
    w = lax.axis_index("s") * NC + lax.axis_index("c")
    lanes = lax.iota(jnp.int32, 16)
    pltpu.sync_copy(begm_hbm.at[pl.ds(w * CPW, 16)], bwin_v)
    pltpu.sync_copy(endm_hbm.at[pl.ds(w * CPW, 16)], ewin_v)
    bm = bwin_v[...]
    em = ewin_v[...]

    acc = jnp.zeros((16,), jnp.float32)
    for j in range(CPW):
        t0 = bm[j]
        t1 = em[j]
        r0 = (w * CPW + j) * RC
        pltpu.sync_copy(user_hbm.at[pl.ds(r0, RC)], uidx_v)
        cp_k = pltpu.async_copy(k_tab_hbm.at[uidx_v], krows_v, sem_k)
        cp_z = pltpu.async_copy(z_hbm.at[pl.ds(r0, RC), :], zrows_v, sem_z)
        cp_k.wait()
        cp_z.wait()
        lo = r0 * F
        hi = lo + RC * F
        t0a = t0 & (-8)
        nb = (t1 - t0a + TB - 1) // TB

        def batch_body(bi, acc):
            wm = t0a + bi * TB
            bs = pl.multiple_of(jnp.minimum(wm, T - TB), 8)
            ci = pltpu.async_copy(seg_hbm.at[pl.ds(bs, TB)], sbuf_v,
                                  sem_i.at[0])
            cf = pltpu.async_copy(feat_hbm.at[pl.ds(bs, TB)], fbuf_v,
                                  sem_i.at[1])
            ci.wait()
            cf.wait()

            def grp(i, acc):
                sv = sbuf_v[pl.ds(i * 16, 16)]
                fv = fbuf_v[pl.ds(i * 16, 16)]
                cv = (sv << 9) | fv
                pos = lanes + (bs + i * 16)
                m = (cv >= lo) & (cv < hi)
                m = m & (pos >= wm) & (pos < wm + TB)
                loc = cv - lo
                r = (loc >> 9) & (RC - 1)
                col = loc & (F - 1)
                kv = plsc.load_gather(krows_v, [r, col], mask=m)
                zv = plsc.load_gather(zrows_v, [r, col], mask=m)
                return acc + jnp.where(m, kv * zv, 0.0)

            return lax.fori_loop(0, TB // 16, grp, acc)

        acc = lax.fori_loop(0, nb, batch_body, acc)

    acc_v[...] = acc
    pltpu.sync_copy(acc_v, out_hbm.at[w])


def kernel(H, G, K, user, item, feat_idx, segment_ids):
    del item
    h_u, tabs = _gather_h(user, H, segment_ids)
    merged = tabs.reshape(NW, 2, TABW).sum(axis=0)
    begm = merged[0]
    endm = merged[1]

    BM = 1024
    z = pl.pallas_call(
        _mm_body,
        grid=(B // BM,),
        in_specs=[
            pl.BlockSpec((BM, D), lambda i: (i, 0)),
            pl.BlockSpec((F, D), lambda i: (0, 0)),
        ],
        out_specs=pl.BlockSpec((BM, F), lambda i: (i, 0)),
        out_shape=jax.ShapeDtypeStruct((B, F), jnp.float32),
    )(h_u, G)

    partials = _main_sc(K, z, user, segment_ids, feat_idx, begm, endm)
    return jnp.sum(partials)


# speedup vs baseline: 1.1585x; 1.0292x over previous
import functools

import jax
import jax.numpy as jnp
from jax import lax
from jax.experimental import pallas as pl
from jax.experimental.pallas import tpu as pltpu
from jax.experimental.pallas import tpu_sc as plsc

B = 16384
F = 512
D = 128
T = 819200
NC = 2
NS = 16
NW = NC * NS
RC = 64
RSH = 6
NCH = B // RC
CPW = NCH // NW
TB = 2048
TPW = T // NW
SSB = 1600
NSB = TPW // SSB
TABW = 272

_mesh = plsc.VectorSubcoreMesh(
    core_axis_name="c", subcore_axis_name="s", num_cores=NC, num_subcores=NS
)
_sc_params = pltpu.CompilerParams(needs_layout_passes=False)


@functools.partial(
    pl.kernel,
    out_type=(
        jax.ShapeDtypeStruct((B, D), jnp.float32),
        jax.ShapeDtypeStruct((NW * 2 * TABW,), jnp.int32),
    ),
    mesh=_mesh,
    compiler_params=_sc_params,
    scratch_types=[
        pltpu.VMEM((2, 128), jnp.int32),
        pltpu.VMEM((2, 128, D), jnp.float32),
        pltpu.VMEM((SSB,), jnp.int32),
        pltpu.VMEM((16,), jnp.int32),
        pltpu.VMEM((TABW,), jnp.int32),
        pltpu.VMEM((TABW,), jnp.int32),
        pltpu.SemaphoreType.DMA((2,)),
    ],
)
def _gather_h(user_hbm, h_tab_hbm, seg_hbm, out_hbm, tabs_hbm,
              idx_v, rows_v, scan_v, pv_v, beg_v, end_v, sem_g):
    w = lax.axis_index("s") * NC + lax.axis_index("c")
    lanes = lax.iota(jnp.int32, 16)
    shl = jnp.maximum(lanes - 1, 0)

    nq = (B // NW) // 128

    def start(q, slot):
        base = w * (B // NW) + q * 128
        pltpu.sync_copy(user_hbm.at[pl.ds(base, 128)], idx_v.at[slot])
        pltpu.async_copy(h_tab_hbm.at[idx_v.at[slot]], rows_v.at[slot],
                         sem_g.at[slot])

    start(0, 0)
    for q in range(nq):
        slot = q % 2
        if q + 1 < nq:
            start(q + 1, 1 - slot)
        base = w * (B // NW) + q * 128
        pltpu.make_async_copy(h_tab_hbm.at[idx_v.at[slot]], rows_v.at[slot],
                              sem_g.at[slot]).wait()
        pltpu.sync_copy(rows_v.at[slot], out_hbm.at[pl.ds(base, 128), :])

    zv = jnp.zeros((16,), jnp.int32)
    for i in range(TABW // 16):
        beg_v[pl.ds(i * 16, 16)] = zv
        end_v[pl.ds(i * 16, 16)] = zv

    p0 = w * TPW
    poff = pl.multiple_of(jnp.maximum(p0 - 16, 0), 16)
    pltpu.sync_copy(seg_hbm.at[pl.ds(poff, 16)], pv_v)
    pvv = pv_v[...]
    prev0 = jnp.where(w == 0, pvv[0], pvv[15]) >> RSH
    full15 = jnp.full((16,), 15, jnp.int32)

    prev_chv = jnp.broadcast_to(prev0, (16,))
    for bi in range(NSB):
        bs = pl.multiple_of(p0 + bi * SSB, 8)
        pltpu.sync_copy(seg_hbm.at[pl.ds(bs, SSB)], scan_v)

        def sgrp(g, pchv):
            sv = scan_v[pl.ds(g * 16, 16)]
            chv = sv >> RSH
            shifted = chv.at[shl].get(mode="promise_in_bounds")
            plast = pchv.at[full15].get(mode="promise_in_bounds")
            left = jnp.where(lanes == 0, plast, shifted)
            mcross = chv != left
            pos = (bs + g * 16) + lanes
            plsc.store_scatter(beg_v, [chv], pos, mask=mcross)
            plsc.store_scatter(end_v, [left], pos, mask=mcross)
            return chv

        prev_chv = lax.fori_loop(0, SSB // 16, sgrp, prev_chv, unroll=4)

    @pl.when(w == NW - 1)
    def _():
        pv = prev_chv.at[full15].get(mode="promise_in_bounds")
        tv = jnp.broadcast_to(jnp.int32(T), (16,))
        plsc.store_scatter(end_v, [pv], tv, mask=lanes == 0)

    pltpu.sync_copy(beg_v, tabs_hbm.at[pl.ds(w * 2 * TABW, TABW)])
    pltpu.sync_copy(end_v, tabs_hbm.at[pl.ds((w * 2 + 1) * TABW, TABW)])


def _mm_body(h_ref, g_ref, o_ref):
    o_ref[...] = lax.dot_general(
        h_ref[...], g_ref[...],
        (((1,), (1,)), ((), ())),
        preferred_element_type=jnp.float32,
    )


@functools.partial(
    pl.kernel,
    out_type=jax.ShapeDtypeStruct((NW, 16), jnp.float32),
    mesh=_mesh,
    compiler_params=_sc_params,
    scratch_types=[
        pltpu.VMEM((16,), jnp.int32),
        pltpu.VMEM((16,), jnp.int32),
        pltpu.VMEM((RC,), jnp.int32),
        pltpu.VMEM((RC, F), jnp.float32),
        pltpu.VMEM((RC, F), jnp.float32),
        pltpu.VMEM((TB,), jnp.int32),
        pltpu.VMEM((TB,), jnp.int32),
        pltpu.VMEM((16,), jnp.float32),
        pltpu.SemaphoreType.DMA,
        pltpu.SemaphoreType.DMA,
        pltpu.SemaphoreType.DMA((2,)),
    ],
)
def _main_sc(k_tab_hbm, z_hbm, user_hbm, seg_hbm, feat_hbm, begm_hbm,
             endm_hbm, out_hbm,
             bwin_v, ewin_v, uidx_v, krows_v, zrows_v, sbuf_v, fbuf_v,
             acc_v, sem_k, sem_z, sem_i):
    w = lax.axis_index("s") * NC + lax.axis_index("c")
    lanes = lax.iota(jnp.int32, 16)
    pltpu.sync_copy(begm_hbm.at[pl.ds(w * CPW, 16)], bwin_v)
    pltpu.sync_copy(endm_hbm.at[pl.ds(w * CPW, 16)], ewin_v)
    bm = bwin_v[...]
    em = ewin_v[...]

    acc = jnp.zeros((16,), jnp.float32)
    for j in range(CPW):
        t0 = bm[j]
        t1 = em[j]
        r0 = (w * CPW + j) * RC
        pltpu.sync_copy(user_hbm.at[pl.ds(r0, RC)], uidx_v)
        cp_k = pltpu.async_copy(k_tab_hbm.at[uidx_v], krows_v, sem_k)
        cp_z = pltpu.async_copy(z_hbm.at[pl.ds(r0, RC), :], zrows_v, sem_z)
        cp_k.wait()
        cp_z.wait()
        lo = r0 * F
        hi = lo + RC * F
        t0a = t0 & (-8)
        nb = (t1 - t0a + TB - 1) // TB

        def batch_body(bi, acc):
            wm = t0a + bi * TB
            bs = pl.multiple_of(jnp.minimum(wm, T - TB), 8)
            ci = pltpu.async_copy(seg_hbm.at[pl.ds(bs, TB)], sbuf_v,
                                  sem_i.at[0])
            cf = pltpu.async_copy(feat_hbm.at[pl.ds(bs, TB)], fbuf_v,
                                  sem_i.at[1])
            ci.wait()
            cf.wait()

            def grp(i, acc):
                sv = sbuf_v[pl.ds(i * 16, 16)]
                fv = fbuf_v[pl.ds(i * 16, 16)]
                cv = (sv << 9) | fv
                pos = lanes + (bs + i * 16)
                m = (cv >= lo) & (cv < hi)
                m = m & (pos >= wm) & (pos < wm + TB)
                loc = cv - lo
                r = (loc >> 9) & (RC - 1)
                col = loc & (F - 1)
                kv = plsc.load_gather(krows_v, [r, col], mask=m)
                zv = plsc.load_gather(zrows_v, [r, col], mask=m)
                return acc + jnp.where(m, kv * zv, 0.0)

            return lax.fori_loop(0, TB // 16, grp, acc)

        acc = lax.fori_loop(0, nb, batch_body, acc)

    acc_v[...] = acc
    pltpu.sync_copy(acc_v, out_hbm.at[w])


def kernel(H, G, K, user, item, feat_idx, segment_ids):
    del item
    h_u, tabs = _gather_h(user, H, segment_ids)
    merged = tabs.reshape(NW, 2, TABW).sum(axis=0)
    begm = merged[0]
    endm = merged[1]

    BM = 2048
    z = pl.pallas_call(
        _mm_body,
        grid=(B // BM,),
        in_specs=[
            pl.BlockSpec((BM, D), lambda i: (i, 0)),
            pl.BlockSpec((F, D), lambda i: (0, 0)),
        ],
        out_specs=pl.BlockSpec((BM, F), lambda i: (i, 0)),
        out_shape=jax.ShapeDtypeStruct((B, F), jnp.float32),
    )(h_u, G)

    partials = _main_sc(K, z, user, segment_ids, feat_idx, begm, endm)
    return jnp.sum(partials)


# speedup vs baseline: 1.2110x; 1.0453x over previous
import functools

import jax
import jax.numpy as jnp
from jax import lax
from jax.experimental import pallas as pl
from jax.experimental.pallas import tpu as pltpu
from jax.experimental.pallas import tpu_sc as plsc

B = 16384
F = 512
D = 128
T = 819200
NC = 2
NS = 16
NW = NC * NS
RC = 64
RSH = 6
NCH = B // RC
CPW = NCH // NW
TB = 2048
TPW = T // NW
SSB = 1600
NSB = TPW // SSB
TABW = 272

_mesh = plsc.VectorSubcoreMesh(
    core_axis_name="c", subcore_axis_name="s", num_cores=NC, num_subcores=NS
)
_sc_params = pltpu.CompilerParams(needs_layout_passes=False)


@functools.partial(
    pl.kernel,
    out_type=(
        jax.ShapeDtypeStruct((B, D), jnp.float32),
        jax.ShapeDtypeStruct((NW * 2 * TABW,), jnp.int32),
    ),
    mesh=_mesh,
    compiler_params=_sc_params,
    scratch_types=[
        pltpu.VMEM((2, 128), jnp.int32),
        pltpu.VMEM((2, 128, D), jnp.float32),
        pltpu.VMEM((SSB,), jnp.int32),
        pltpu.VMEM((16,), jnp.int32),
        pltpu.VMEM((TABW,), jnp.int32),
        pltpu.VMEM((TABW,), jnp.int32),
        pltpu.SemaphoreType.DMA((2,)),
    ],
)
def _gather_h(user_hbm, h_tab_hbm, seg_hbm, out_hbm, tabs_hbm,
              idx_v, rows_v, scan_v, pv_v, beg_v, end_v, sem_g):
    w = lax.axis_index("s") * NC + lax.axis_index("c")
    lanes = lax.iota(jnp.int32, 16)
    shl = jnp.maximum(lanes - 1, 0)

    nq = (B // NW) // 128

    def start(q, slot):
        base = w * (B // NW) + q * 128
        pltpu.sync_copy(user_hbm.at[pl.ds(base, 128)], idx_v.at[slot])
        pltpu.async_copy(h_tab_hbm.at[idx_v.at[slot]], rows_v.at[slot],
                         sem_g.at[slot])

    start(0, 0)
    for q in range(nq):
        slot = q % 2
        if q + 1 < nq:
            start(q + 1, 1 - slot)
        base = w * (B // NW) + q * 128
        pltpu.make_async_copy(h_tab_hbm.at[idx_v.at[slot]], rows_v.at[slot],
                              sem_g.at[slot]).wait()
        pltpu.sync_copy(rows_v.at[slot], out_hbm.at[pl.ds(base, 128), :])

    zv = jnp.zeros((16,), jnp.int32)
    for i in range(TABW // 16):
        beg_v[pl.ds(i * 16, 16)] = zv
        end_v[pl.ds(i * 16, 16)] = zv

    p0 = w * TPW
    poff = pl.multiple_of(jnp.maximum(p0 - 16, 0), 16)
    pltpu.sync_copy(seg_hbm.at[pl.ds(poff, 16)], pv_v)
    pvv = pv_v[...]
    prev0 = jnp.where(w == 0, pvv[0], pvv[15]) >> RSH
    full15 = jnp.full((16,), 15, jnp.int32)

    prev_chv = jnp.broadcast_to(prev0, (16,))
    for bi in range(NSB):
        bs = pl.multiple_of(p0 + bi * SSB, 8)
        pltpu.sync_copy(seg_hbm.at[pl.ds(bs, SSB)], scan_v)

        def sgrp(g, pchv):
            sv = scan_v[pl.ds(g * 16, 16)]
            chv = sv >> RSH
            shifted = chv.at[shl].get(mode="promise_in_bounds")
            plast = pchv.at[full15].get(mode="promise_in_bounds")
            left = jnp.where(lanes == 0, plast, shifted)
            mcross = chv != left
            pos = (bs + g * 16) + lanes
            plsc.store_scatter(beg_v, [chv], pos, mask=mcross)
            plsc.store_scatter(end_v, [left], pos, mask=mcross)
            return chv

        prev_chv = lax.fori_loop(0, SSB // 16, sgrp, prev_chv, unroll=4)

    @pl.when(w == NW - 1)
    def _():
        pv = prev_chv.at[full15].get(mode="promise_in_bounds")
        tv = jnp.broadcast_to(jnp.int32(T), (16,))
        plsc.store_scatter(end_v, [pv], tv, mask=lanes == 0)

    pltpu.sync_copy(beg_v, tabs_hbm.at[pl.ds(w * 2 * TABW, TABW)])
    pltpu.sync_copy(end_v, tabs_hbm.at[pl.ds((w * 2 + 1) * TABW, TABW)])


def _mm_body(h_ref, g_ref, o_ref):
    o_ref[...] = lax.dot_general(
        h_ref[...], g_ref[...],
        (((1,), (1,)), ((), ())),
        preferred_element_type=jnp.float32,
    )


@functools.partial(
    pl.kernel,
    out_type=jax.ShapeDtypeStruct((NW, 16), jnp.float32),
    mesh=_mesh,
    compiler_params=_sc_params,
    scratch_types=[
        pltpu.VMEM((16,), jnp.int32),
        pltpu.VMEM((16,), jnp.int32),
        pltpu.VMEM((CPW * RC,), jnp.int32),
        pltpu.VMEM((2, RC, F), jnp.float32),
        pltpu.VMEM((RC, F), jnp.float32),
        pltpu.VMEM((TB,), jnp.int32),
        pltpu.VMEM((TB,), jnp.int32),
        pltpu.VMEM((16,), jnp.float32),
        pltpu.SemaphoreType.DMA((2,)),
        pltpu.SemaphoreType.DMA,
        pltpu.SemaphoreType.DMA((2,)),
    ],
)
def _main_sc(k_tab_hbm, z_hbm, user_hbm, seg_hbm, feat_hbm, begm_hbm,
             endm_hbm, out_hbm,
             bwin_v, ewin_v, uidx_v, krows2_v, zrows_v, sbuf_v, fbuf_v,
             acc_v, sem_k, sem_z, sem_i):
    w = lax.axis_index("s") * NC + lax.axis_index("c")
    lanes = lax.iota(jnp.int32, 16)
    pltpu.sync_copy(user_hbm.at[pl.ds(w * CPW * RC, CPW * RC)], uidx_v)
    pltpu.async_copy(k_tab_hbm.at[uidx_v.at[pl.ds(0, RC)]],
                     krows2_v.at[0], sem_k.at[0])
    pltpu.sync_copy(begm_hbm.at[pl.ds(w * CPW, 16)], bwin_v)
    pltpu.sync_copy(endm_hbm.at[pl.ds(w * CPW, 16)], ewin_v)
    bm = bwin_v[...]
    em = ewin_v[...]

    acc = jnp.zeros((16,), jnp.float32)
    for j in range(CPW):
        slot = j % 2
        t0 = bm[j]
        t1 = em[j]
        r0 = (w * CPW + j) * RC
        cp_z = pltpu.async_copy(z_hbm.at[pl.ds(r0, RC), :], zrows_v, sem_z)
        pltpu.make_async_copy(k_tab_hbm.at[uidx_v.at[pl.ds(j * RC, RC)]],
                              krows2_v.at[slot], sem_k.at[slot]).wait()
        if j + 1 < CPW:
            pltpu.async_copy(
                k_tab_hbm.at[uidx_v.at[pl.ds((j + 1) * RC, RC)]],
                krows2_v.at[1 - slot], sem_k.at[1 - slot])
        cp_z.wait()
        krows_v = krows2_v.at[slot]
        lo = r0 * F
        hi = lo + RC * F
        t0a = t0 & (-8)
        nb = (t1 - t0a + TB - 1) // TB

        def batch_body(bi, acc):
            wm = t0a + bi * TB
            bs = pl.multiple_of(jnp.minimum(wm, T - TB), 8)
            ci = pltpu.async_copy(seg_hbm.at[pl.ds(bs, TB)], sbuf_v,
                                  sem_i.at[0])
            cf = pltpu.async_copy(feat_hbm.at[pl.ds(bs, TB)], fbuf_v,
                                  sem_i.at[1])
            ci.wait()
            cf.wait()

            def grp(i, acc):
                sv = sbuf_v[pl.ds(i * 16, 16)]
                fv = fbuf_v[pl.ds(i * 16, 16)]
                cv = (sv << 9) | fv
                pos = lanes + (bs + i * 16)
                m = (cv >= lo) & (cv < hi)
                m = m & (pos >= wm) & (pos < wm + TB)
                loc = cv - lo
                r = (loc >> 9) & (RC - 1)
                col = loc & (F - 1)
                kv = plsc.load_gather(krows_v, [r, col], mask=m)
                zv = plsc.load_gather(zrows_v, [r, col], mask=m)
                return acc + jnp.where(m, kv * zv, 0.0)

            return lax.fori_loop(0, TB // 16, grp, acc)

        acc = lax.fori_loop(0, nb, batch_body, acc)

    acc_v[...] = acc
    pltpu.sync_copy(acc_v, out_hbm.at[w])


def kernel(H, G, K, user, item, feat_idx, segment_ids):
    del item
    h_u, tabs = _gather_h(user, H, segment_ids)
    merged = tabs.reshape(NW, 2, TABW).sum(axis=0)
    begm = merged[0]
    endm = merged[1]

    BM = 2048
    z = pl.pallas_call(
        _mm_body,
        grid=(B // BM,),
        in_specs=[
            pl.BlockSpec((BM, D), lambda i: (i, 0)),
            pl.BlockSpec((F, D), lambda i: (0, 0)),
        ],
        out_specs=pl.BlockSpec((BM, F), lambda i: (i, 0)),
        out_shape=jax.ShapeDtypeStruct((B, F), jnp.float32),
    )(h_u, G)

    partials = _main_sc(K, z, user, segment_ids, feat_idx, begm, endm)
    return jnp.sum(partials)


# speedup vs baseline: 1.2929x; 1.0677x over previous
import functools

import jax
import jax.numpy as jnp
from jax import lax
from jax.experimental import pallas as pl
from jax.experimental.pallas import tpu as pltpu
from jax.experimental.pallas import tpu_sc as plsc

B = 16384
F = 512
D = 128
T = 819200
NC = 2
NS = 16
NW = NC * NS
RC = 64
RSH = 6
NCH = B // RC
CPW = NCH // NW
TB = 2048
TPW = T // NW
SSB = 3200
NSB = TPW // SSB
TABW = 272

_mesh = plsc.VectorSubcoreMesh(
    core_axis_name="c", subcore_axis_name="s", num_cores=NC, num_subcores=NS
)
_sc_params = pltpu.CompilerParams(needs_layout_passes=False)


@functools.partial(
    pl.kernel,
    out_type=(
        jax.ShapeDtypeStruct((B, D), jnp.float32),
        jax.ShapeDtypeStruct((NW * 2 * TABW,), jnp.int32),
    ),
    mesh=_mesh,
    compiler_params=_sc_params,
    scratch_types=[
        pltpu.VMEM((2, 128), jnp.int32),
        pltpu.VMEM((2, 128, D), jnp.float32),
        pltpu.VMEM((2, SSB), jnp.int32),
        pltpu.VMEM((16,), jnp.int32),
        pltpu.VMEM((TABW,), jnp.int32),
        pltpu.VMEM((TABW,), jnp.int32),
        pltpu.SemaphoreType.DMA((2,)),
        pltpu.SemaphoreType.DMA((2,)),
        pltpu.SemaphoreType.DMA((2,)),
    ],
)
def _gather_h(user_hbm, h_tab_hbm, seg_hbm, out_hbm, tabs_hbm,
              idx_v, rows_v, scan_v, pv_v, beg_v, end_v, sem_g, sem_o,
              sem_s):
    w = lax.axis_index("s") * NC + lax.axis_index("c")
    lanes = lax.iota(jnp.int32, 16)
    shl = jnp.maximum(lanes - 1, 0)

    nq = (B // NW) // 128

    def start(q, slot):
        base = w * (B // NW) + q * 128
        pltpu.sync_copy(user_hbm.at[pl.ds(base, 128)], idx_v.at[slot])
        pltpu.async_copy(h_tab_hbm.at[idx_v.at[slot]], rows_v.at[slot],
                         sem_g.at[slot])

    def obase(q):
        return w * (B // NW) + q * 128

    start(0, 0)
    for q in range(nq):
        slot = q % 2
        if q + 1 < nq:
            if q >= 1:
                pltpu.make_async_copy(
                    rows_v.at[1 - slot],
                    out_hbm.at[pl.ds(obase(q - 1), 128), :],
                    sem_o.at[1 - slot]).wait()
            start(q + 1, 1 - slot)
        pltpu.make_async_copy(h_tab_hbm.at[idx_v.at[slot]], rows_v.at[slot],
                              sem_g.at[slot]).wait()
        pltpu.async_copy(rows_v.at[slot], out_hbm.at[pl.ds(obase(q), 128), :],
                         sem_o.at[slot])
    for q in range(nq - 2, nq):
        pltpu.make_async_copy(rows_v.at[q % 2],
                              out_hbm.at[pl.ds(obase(q), 128), :],
                              sem_o.at[q % 2]).wait()

    zv = jnp.zeros((16,), jnp.int32)
    for i in range(TABW // 16):
        beg_v[pl.ds(i * 16, 16)] = zv
        end_v[pl.ds(i * 16, 16)] = zv

    p0 = w * TPW
    poff = pl.multiple_of(jnp.maximum(p0 - 16, 0), 16)
    pltpu.sync_copy(seg_hbm.at[pl.ds(poff, 16)], pv_v)
    pvv = pv_v[...]
    prev0 = jnp.where(w == 0, pvv[0], pvv[15]) >> RSH
    full15 = jnp.full((16,), 15, jnp.int32)

    def sbs(bi):
        return pl.multiple_of(p0 + bi * SSB, 8)

    pltpu.async_copy(seg_hbm.at[pl.ds(sbs(0), SSB)], scan_v.at[0],
                     sem_s.at[0])
    prev_chv = jnp.broadcast_to(prev0, (16,))
    for bi in range(NSB):
        sslot = bi % 2
        if bi + 1 < NSB:
            pltpu.async_copy(seg_hbm.at[pl.ds(sbs(bi + 1), SSB)],
                             scan_v.at[1 - sslot], sem_s.at[1 - sslot])
        bs = sbs(bi)
        pltpu.make_async_copy(seg_hbm.at[pl.ds(bs, SSB)], scan_v.at[sslot],
                              sem_s.at[sslot]).wait()

        def sgrp(g, pchv):
            sv = scan_v[sslot, pl.ds(g * 16, 16)]
            chv = sv >> RSH
            shifted = chv.at[shl].get(mode="promise_in_bounds")
            plast = pchv.at[full15].get(mode="promise_in_bounds")
            left = jnp.where(lanes == 0, plast, shifted)
            mcross = chv != left
            pos = (bs + g * 16) + lanes
            plsc.store_scatter(beg_v, [chv], pos, mask=mcross)
            plsc.store_scatter(end_v, [left], pos, mask=mcross)
            return chv

        prev_chv = lax.fori_loop(0, SSB // 16, sgrp, prev_chv, unroll=4)

    @pl.when(w == NW - 1)
    def _():
        pv = prev_chv.at[full15].get(mode="promise_in_bounds")
        tv = jnp.broadcast_to(jnp.int32(T), (16,))
        plsc.store_scatter(end_v, [pv], tv, mask=lanes == 0)

    pltpu.sync_copy(beg_v, tabs_hbm.at[pl.ds(w * 2 * TABW, TABW)])
    pltpu.sync_copy(end_v, tabs_hbm.at[pl.ds((w * 2 + 1) * TABW, TABW)])


def _mm_body(h_ref, g_ref, o_ref):
    o_ref[...] = lax.dot_general(
        h_ref[...], g_ref[...],
        (((1,), (1,)), ((), ())),
        preferred_element_type=jnp.float32,
    )


@functools.partial(
    pl.kernel,
    out_type=jax.ShapeDtypeStruct((NW, 16), jnp.float32),
    mesh=_mesh,
    compiler_params=_sc_params,
    scratch_types=[
        pltpu.VMEM((16,), jnp.int32),
        pltpu.VMEM((16,), jnp.int32),
        pltpu.VMEM((CPW * RC,), jnp.int32),
        pltpu.VMEM((2, RC, F), jnp.float32),
        pltpu.VMEM((RC, F), jnp.float32),
        pltpu.VMEM((TB,), jnp.int32),
        pltpu.VMEM((TB,), jnp.int32),
        pltpu.VMEM((16,), jnp.float32),
        pltpu.SemaphoreType.DMA((2,)),
        pltpu.SemaphoreType.DMA,
        pltpu.SemaphoreType.DMA((2,)),
    ],
)
def _main_sc(k_tab_hbm, z_hbm, user_hbm, seg_hbm, feat_hbm, begm_hbm,
             endm_hbm, out_hbm,
             bwin_v, ewin_v, uidx_v, krows2_v, zrows_v, sbuf_v, fbuf_v,
             acc_v, sem_k, sem_z, sem_i):
    w = lax.axis_index("s") * NC + lax.axis_index("c")
    lanes = lax.iota(jnp.int32, 16)
    pltpu.sync_copy(user_hbm.at[pl.ds(w * CPW * RC, CPW * RC)], uidx_v)
    pltpu.async_copy(k_tab_hbm.at[uidx_v.at[pl.ds(0, RC)]],
                     krows2_v.at[0], sem_k.at[0])
    pltpu.sync_copy(begm_hbm.at[pl.ds(w * CPW, 16)], bwin_v)
    pltpu.sync_copy(endm_hbm.at[pl.ds(w * CPW, 16)], ewin_v)
    bm = bwin_v[...]
    em = ewin_v[...]

    acc = jnp.zeros((16,), jnp.float32)
    for j in range(CPW):
        slot = j % 2
        t0 = bm[j]
        t1 = em[j]
        r0 = (w * CPW + j) * RC
        cp_z = pltpu.async_copy(z_hbm.at[pl.ds(r0, RC), :], zrows_v, sem_z)
        pltpu.make_async_copy(k_tab_hbm.at[uidx_v.at[pl.ds(j * RC, RC)]],
                              krows2_v.at[slot], sem_k.at[slot]).wait()
        if j + 1 < CPW:
            pltpu.async_copy(
                k_tab_hbm.at[uidx_v.at[pl.ds((j + 1) * RC, RC)]],
                krows2_v.at[1 - slot], sem_k.at[1 - slot])
        cp_z.wait()
        krows_v = krows2_v.at[slot]
        lo = r0 * F
        hi = lo + RC * F
        t0a = t0 & (-8)
        nb = (t1 - t0a + TB - 1) // TB

        def batch_body(bi, acc):
            wm = t0a + bi * TB
            bs = pl.multiple_of(jnp.minimum(wm, T - TB), 8)
            ci = pltpu.async_copy(seg_hbm.at[pl.ds(bs, TB)], sbuf_v,
                                  sem_i.at[0])
            cf = pltpu.async_copy(feat_hbm.at[pl.ds(bs, TB)], fbuf_v,
                                  sem_i.at[1])
            ci.wait()
            cf.wait()

            def grp(i, acc):
                sv = sbuf_v[pl.ds(i * 16, 16)]
                fv = fbuf_v[pl.ds(i * 16, 16)]
                cv = (sv << 9) | fv
                pos = lanes + (bs + i * 16)
                m = (cv >= lo) & (cv < hi)
                m = m & (pos >= wm) & (pos < wm + TB)
                loc = cv - lo
                r = (loc >> 9) & (RC - 1)
                col = loc & (F - 1)
                kv = plsc.load_gather(krows_v, [r, col], mask=m)
                zv = plsc.load_gather(zrows_v, [r, col], mask=m)
                return acc + jnp.where(m, kv * zv, 0.0)

            return lax.fori_loop(0, TB // 16, grp, acc)

        acc = lax.fori_loop(0, nb, batch_body, acc)

    acc_v[...] = acc
    pltpu.sync_copy(acc_v, out_hbm.at[w])


def kernel(H, G, K, user, item, feat_idx, segment_ids):
    del item
    h_u, tabs = _gather_h(user, H, segment_ids)
    merged = tabs.reshape(NW, 2, TABW).sum(axis=0)
    begm = merged[0]
    endm = merged[1]

    BM = 2048
    z = pl.pallas_call(
        _mm_body,
        grid=(B // BM,),
        in_specs=[
            pl.BlockSpec((BM, D), lambda i: (i, 0)),
            pl.BlockSpec((F, D), lambda i: (0, 0)),
        ],
        out_specs=pl.BlockSpec((BM, F), lambda i: (i, 0)),
        out_shape=jax.ShapeDtypeStruct((B, F), jnp.float32),
    )(h_u, G)

    partials = _main_sc(K, z, user, segment_ids, feat_idx, begm, endm)
    return jnp.sum(partials)


# speedup vs baseline: 1.3019x; 1.0069x over previous
import functools

import jax
import jax.numpy as jnp
from jax import lax
from jax.experimental import pallas as pl
from jax.experimental.pallas import tpu as pltpu
from jax.experimental.pallas import tpu_sc as plsc

B = 16384
F = 512
D = 128
T = 819200
NC = 2
NS = 16
NW = NC * NS
RC = 64
RSH = 6
NCH = B // RC
CPW = NCH // NW
TB = 2048
TPW = T // NW
SSB = 3200
NSB = TPW // SSB
TABW = 272

_mesh = plsc.VectorSubcoreMesh(
    core_axis_name="c", subcore_axis_name="s", num_cores=NC, num_subcores=NS
)
_sc_params = pltpu.CompilerParams(needs_layout_passes=False)


@functools.partial(
    pl.kernel,
    out_type=(
        jax.ShapeDtypeStruct((B, D), jnp.float32),
        jax.ShapeDtypeStruct((NW * 2 * TABW,), jnp.int32),
    ),
    mesh=_mesh,
    compiler_params=_sc_params,
    scratch_types=[
        pltpu.VMEM((2, 128), jnp.int32),
        pltpu.VMEM((2, 128, D), jnp.float32),
        pltpu.VMEM((2, SSB), jnp.int32),
        pltpu.VMEM((16,), jnp.int32),
        pltpu.VMEM((TABW,), jnp.int32),
        pltpu.VMEM((TABW,), jnp.int32),
        pltpu.SemaphoreType.DMA((2,)),
        pltpu.SemaphoreType.DMA((2,)),
        pltpu.SemaphoreType.DMA((2,)),
    ],
)
def _gather_h(user_hbm, h_tab_hbm, seg_hbm, out_hbm, tabs_hbm,
              idx_v, rows_v, scan_v, pv_v, beg_v, end_v, sem_g, sem_o,
              sem_s):
    w = lax.axis_index("s") * NC + lax.axis_index("c")
    lanes = lax.iota(jnp.int32, 16)
    shl = jnp.maximum(lanes - 1, 0)

    nq = (B // NW) // 128

    def start(q, slot):
        base = w * (B // NW) + q * 128
        pltpu.sync_copy(user_hbm.at[pl.ds(base, 128)], idx_v.at[slot])
        pltpu.async_copy(h_tab_hbm.at[idx_v.at[slot]], rows_v.at[slot],
                         sem_g.at[slot])

    def obase(q):
        return w * (B // NW) + q * 128

    start(0, 0)
    for q in range(nq):
        slot = q % 2
        if q + 1 < nq:
            if q >= 1:
                pltpu.make_async_copy(
                    rows_v.at[1 - slot],
                    out_hbm.at[pl.ds(obase(q - 1), 128), :],
                    sem_o.at[1 - slot]).wait()
            start(q + 1, 1 - slot)
        pltpu.make_async_copy(h_tab_hbm.at[idx_v.at[slot]], rows_v.at[slot],
                              sem_g.at[slot]).wait()
        pltpu.async_copy(rows_v.at[slot], out_hbm.at[pl.ds(obase(q), 128), :],
                         sem_o.at[slot])
    for q in range(nq - 2, nq):
        pltpu.make_async_copy(rows_v.at[q % 2],
                              out_hbm.at[pl.ds(obase(q), 128), :],
                              sem_o.at[q % 2]).wait()

    zv = jnp.zeros((16,), jnp.int32)
    for i in range(TABW // 16):
        beg_v[pl.ds(i * 16, 16)] = zv
        end_v[pl.ds(i * 16, 16)] = zv

    p0 = w * TPW
    poff = pl.multiple_of(jnp.maximum(p0 - 16, 0), 16)
    pltpu.sync_copy(seg_hbm.at[pl.ds(poff, 16)], pv_v)
    pvv = pv_v[...]
    prev0 = jnp.where(w == 0, pvv[0], pvv[15]) >> RSH
    full15 = jnp.full((16,), 15, jnp.int32)

    def sbs(bi):
        return pl.multiple_of(p0 + bi * SSB, 8)

    pltpu.async_copy(seg_hbm.at[pl.ds(sbs(0), SSB)], scan_v.at[0],
                     sem_s.at[0])
    prev_chv = jnp.broadcast_to(prev0, (16,))
    for bi in range(NSB):
        sslot = bi % 2
        if bi + 1 < NSB:
            pltpu.async_copy(seg_hbm.at[pl.ds(sbs(bi + 1), SSB)],
                             scan_v.at[1 - sslot], sem_s.at[1 - sslot])
        bs = sbs(bi)
        pltpu.make_async_copy(seg_hbm.at[pl.ds(bs, SSB)], scan_v.at[sslot],
                              sem_s.at[sslot]).wait()

        def sgrp(g, pchv):
            sv = scan_v[sslot, pl.ds(g * 16, 16)]
            chv = sv >> RSH
            shifted = chv.at[shl].get(mode="promise_in_bounds")
            plast = pchv.at[full15].get(mode="promise_in_bounds")
            left = jnp.where(lanes == 0, plast, shifted)
            mcross = chv != left
            pos = (bs + g * 16) + lanes
            plsc.store_scatter(beg_v, [chv], pos, mask=mcross)
            plsc.store_scatter(end_v, [left], pos, mask=mcross)
            return chv

        prev_chv = lax.fori_loop(0, SSB // 16, sgrp, prev_chv, unroll=4)

    @pl.when(w == NW - 1)
    def _():
        pv = prev_chv.at[full15].get(mode="promise_in_bounds")
        tv = jnp.broadcast_to(jnp.int32(T), (16,))
        plsc.store_scatter(end_v, [pv], tv, mask=lanes == 0)

    pltpu.sync_copy(beg_v, tabs_hbm.at[pl.ds(w * 2 * TABW, TABW)])
    pltpu.sync_copy(end_v, tabs_hbm.at[pl.ds((w * 2 + 1) * TABW, TABW)])


def _mm_body(h_ref, g_ref, o_ref):
    o_ref[...] = lax.dot_general(
        h_ref[...], g_ref[...],
        (((1,), (1,)), ((), ())),
        preferred_element_type=jnp.float32,
    )


@functools.partial(
    pl.kernel,
    out_type=jax.ShapeDtypeStruct((NW, 16), jnp.float32),
    mesh=_mesh,
    compiler_params=_sc_params,
    scratch_types=[
        pltpu.VMEM((16,), jnp.int32),
        pltpu.VMEM((16,), jnp.int32),
        pltpu.VMEM((CPW * RC,), jnp.int32),
        pltpu.VMEM((2, RC, F), jnp.float32),
        pltpu.VMEM((RC, F), jnp.float32),
        pltpu.VMEM((TB,), jnp.int32),
        pltpu.VMEM((TB,), jnp.int32),
        pltpu.VMEM((16,), jnp.float32),
        pltpu.SemaphoreType.DMA((2,)),
        pltpu.SemaphoreType.DMA,
        pltpu.SemaphoreType.DMA((2,)),
    ],
)
def _main_sc(k_tab_hbm, z_hbm, user_hbm, seg_hbm, feat_hbm, begm_hbm,
             endm_hbm, out_hbm,
             bwin_v, ewin_v, uidx_v, krows2_v, zrows_v, sbuf_v, fbuf_v,
             acc_v, sem_k, sem_z, sem_i):
    w = lax.axis_index("s") * NC + lax.axis_index("c")
    lanes = lax.iota(jnp.int32, 16)
    pltpu.sync_copy(user_hbm.at[pl.ds(w * CPW * RC, CPW * RC)], uidx_v)
    pltpu.async_copy(k_tab_hbm.at[uidx_v.at[pl.ds(0, RC)]],
                     krows2_v.at[0], sem_k.at[0])
    pltpu.sync_copy(begm_hbm.at[pl.ds(w * CPW, 16)], bwin_v)
    pltpu.sync_copy(endm_hbm.at[pl.ds(w * CPW, 16)], ewin_v)
    bm = bwin_v[...]
    em = ewin_v[...]

    acc = jnp.zeros((16,), jnp.float32)
    for j in range(CPW):
        slot = j % 2
        t0 = bm[j]
        t1 = em[j]
        r0 = (w * CPW + j) * RC
        cp_z = pltpu.async_copy(z_hbm.at[pl.ds(r0, RC), :], zrows_v, sem_z)
        pltpu.make_async_copy(k_tab_hbm.at[uidx_v.at[pl.ds(j * RC, RC)]],
                              krows2_v.at[slot], sem_k.at[slot]).wait()
        if j + 1 < CPW:
            pltpu.async_copy(
                k_tab_hbm.at[uidx_v.at[pl.ds((j + 1) * RC, RC)]],
                krows2_v.at[1 - slot], sem_k.at[1 - slot])
        cp_z.wait()
        krows_v = krows2_v.at[slot]
        lo = r0 * F
        hi = lo + RC * F
        t0a = t0 & (-8)
        nb = (t1 - t0a + TB - 1) // TB
        bs0 = pl.multiple_of(jnp.minimum(t0a, T - TB), 8)

        @pl.when(nb > 0)
        def _():
            pltpu.async_copy(seg_hbm.at[pl.ds(bs0, TB)], sbuf_v, sem_i.at[0])
            pltpu.async_copy(feat_hbm.at[pl.ds(bs0, TB)], fbuf_v,
                             sem_i.at[1])

        def batch_body(bi, acc):
            wm = t0a + bi * TB
            bs = pl.multiple_of(jnp.minimum(wm, T - TB), 8)

            @pl.when(bi > 0)
            def _():
                pltpu.async_copy(seg_hbm.at[pl.ds(bs, TB)], sbuf_v,
                                 sem_i.at[0])
                pltpu.async_copy(feat_hbm.at[pl.ds(bs, TB)], fbuf_v,
                                 sem_i.at[1])

            pltpu.make_async_copy(seg_hbm.at[pl.ds(bs, TB)], sbuf_v,
                                  sem_i.at[0]).wait()
            pltpu.make_async_copy(feat_hbm.at[pl.ds(bs, TB)], fbuf_v,
                                  sem_i.at[1]).wait()

            def grp(i, acc):
                sv = sbuf_v[pl.ds(i * 16, 16)]
                fv = fbuf_v[pl.ds(i * 16, 16)]
                cv = (sv << 9) | fv
                pos = lanes + (bs + i * 16)
                m = (cv >= lo) & (cv < hi)
                m = m & (pos >= wm) & (pos < wm + TB)
                loc = cv - lo
                r = (loc >> 9) & (RC - 1)
                col = loc & (F - 1)
                kv = plsc.load_gather(krows_v, [r, col], mask=m)
                zv = plsc.load_gather(zrows_v, [r, col], mask=m)
                return acc + jnp.where(m, kv * zv, 0.0)

            return lax.fori_loop(0, TB // 16, grp, acc)

        acc = lax.fori_loop(0, nb, batch_body, acc)

    acc_v[...] = acc
    pltpu.sync_copy(acc_v, out_hbm.at[w])


def kernel(H, G, K, user, item, feat_idx, segment_ids):
    del item
    h_u, tabs = _gather_h(user, H, segment_ids)
    merged = tabs.reshape(NW, 2, TABW).sum(axis=0)
    begm = merged[0]
    endm = merged[1]

    BM = 4096
    z = pl.pallas_call(
        _mm_body,
        grid=(B // BM,),
        in_specs=[
            pl.BlockSpec((BM, D), lambda i: (i, 0)),
            pl.BlockSpec((F, D), lambda i: (0, 0)),
        ],
        out_specs=pl.BlockSpec((BM, F), lambda i: (i, 0)),
        out_shape=jax.ShapeDtypeStruct((B, F), jnp.float32),
    )(h_u, G)

    partials = _main_sc(K, z, user, segment_ids, feat_idx, begm, endm)
    return jnp.sum(partials)
